# baseline (speedup 1.0000x reference)
import jax
import jax.numpy as jnp
from jax import lax
from jax.experimental import pallas as pl
from jax.experimental.pallas import tpu as pltpu
from jax.experimental.pallas import tpu_sc as plsc

N = 10000
E = 320000
D = 128

NC = 2
NS = 16
CHUNK = 48
N_PAD = 10240
HALF = N_PAD // 2
E_PAD = 331776
IDX_ROWS = E_PAD // CHUNK
OUTER = IDX_ROWS // NS // 8
ACC_ROWS = HALF + 1
DUMP = HALF


def _agg_body(x_hbm, src_hbm, dst_hbm, out_hbm,
              xs, acc, srcb, dstb, stag, sem):
    c = lax.axis_index("c")
    s = lax.axis_index("s")
    last = s == NS - 1

    @pl.when(jnp.logical_not(last))
    def _():
        pltpu.sync_copy(x_hbm.at[pl.ds(s * 632, 632)],
                        xs.at[pl.ds(s * 632, 632)])

    @pl.when(last)
    def _():
        pltpu.sync_copy(x_hbm.at[pl.ds(9480, 520)], xs.at[pl.ds(9480, 520)])

    short = (c == 1) & last
    a0 = s * 320

    @pl.when(jnp.logical_not(short))
    def _():
        pltpu.sync_copy(x_hbm.at[pl.ds(c * HALF + a0, 320)],
                        acc.at[pl.ds(a0, 320)])

    @pl.when(short)
    def _():
        pltpu.sync_copy(x_hbm.at[pl.ds(9920, 80)], acc.at[pl.ds(4800, 80)])

    plsc.subcore_barrier()

    def outer(i, carry):
        rb = s * (IDX_ROWS // NS) + i * 8
        pltpu.sync_copy(src_hbm.at[pl.ds(rb, 8)], srcb)
        pltpu.sync_copy(dst_hbm.at[pl.ds(rb, 8)], dstb)
        for r in range(8):
            for g in range(3):
                sl = pl.ds(g * 16, 16)
                dl = dstb[r, sl] - c * HALF
                m = (dl >= 0) & (dl < HALF)
                dstb[r, sl] = jnp.where(m, dl, DUMP)
        for r in range(8):
            pltpu.async_copy(xs.at[srcb.at[r]], stag, sem).wait()
            pltpu.sync_copy(stag, acc.at[dstb.at[r]], add=True)
        return carry

    lax.fori_loop(0, OUTER, outer, 0)
    plsc.subcore_barrier()
    pltpu.sync_copy(acc.at[pl.ds(a0, 320)],
                    out_hbm.at[pl.ds(c * HALF + a0, 320)])


_agg = pl.kernel(
    _agg_body,
    out_type=jax.ShapeDtypeStruct((N_PAD, D), jnp.float32),
    mesh=plsc.VectorSubcoreMesh(core_axis_name="c", subcore_axis_name="s"),
    scratch_types=[
        pltpu.VMEM_SHARED((N, D), jnp.float32),
        pltpu.VMEM_SHARED((ACC_ROWS, D), jnp.float32),
        pltpu.VMEM((8, CHUNK), jnp.int32),
        pltpu.VMEM((8, CHUNK), jnp.int32),
        pltpu.VMEM((CHUNK, D), jnp.float32),
        pltpu.SemaphoreType.DMA,
    ],
)

BLK = 1000
GRID = N // BLK


def _mlp_body(h_ref, W1_ref, b1_ref, W2_ref, b2_ref, out_ref):
    h = jnp.dot(h_ref[:], W1_ref[:], preferred_element_type=jnp.float32)
    h = jnp.maximum(h + b1_ref[:], 0.0)
    h = jnp.dot(h, W2_ref[:], preferred_element_type=jnp.float32) + b2_ref[:]
    out_ref[:] = jnp.maximum(h, 0.0)


def _mlp2_body(h_ref, W1_ref, b1_ref, W2_ref, b2_ref,
               out_ref, sum_ref, sumsq_ref):
    h = jnp.dot(h_ref[:], W1_ref[:], preferred_element_type=jnp.float32)
    h = jnp.maximum(h + b1_ref[:], 0.0)
    h = jnp.dot(h, W2_ref[:], preferred_element_type=jnp.float32) + b2_ref[:]
    z = jnp.maximum(h, 0.0)
    out_ref[:] = z
    ps = jnp.sum(z, axis=0, keepdims=True)
    pq = jnp.sum(z * z, axis=0, keepdims=True)

    @pl.when(pl.program_id(0) == 0)
    def _():
        sum_ref[:] = ps
        sumsq_ref[:] = pq

    @pl.when(pl.program_id(0) != 0)
    def _():
        sum_ref[:] = sum_ref[:] + ps
        sumsq_ref[:] = sumsq_ref[:] + pq


def _bn_proj_body(z_ref, sum_ref, sumsq_ref, gamma_ref, beta_ref,
                  pW_ref, pb_ref, a_ref, zo_ref, p_ref):
    mean = sum_ref[:] / N
    var = sumsq_ref[:] / N - mean * mean
    inv = lax.rsqrt(var + 1e-5)
    zn = (z_ref[:] - mean) * (inv * gamma_ref[:]) + beta_ref[:]
    zo_ref[:] = zn
    p = jnp.dot(zn, pW_ref[:], preferred_element_type=jnp.float32) + pb_ref[:]
    p_ref[:] = jnp.where(p >= 0.0, p, a_ref[0, 0] * p)


def _row_spec():
    return pl.BlockSpec((BLK, D), lambda i: (i, 0))


def _full_spec(shape):
    nd = len(shape)
    return pl.BlockSpec(shape, lambda i: (0,) * nd)


def _mlp(h, W1, b1, W2, b2, with_stats):
    in_specs = [
        _row_spec(),
        _full_spec((D, D)),
        _full_spec((1, D)),
        _full_spec((D, D)),
        _full_spec((1, D)),
    ]
    if with_stats:
        return pl.pallas_call(
            _mlp2_body,
            grid=(GRID,),
            in_specs=in_specs,
            out_specs=[_row_spec(), _full_spec((1, D)), _full_spec((1, D))],
            out_shape=[
                jax.ShapeDtypeStruct((N, D), jnp.float32),
                jax.ShapeDtypeStruct((1, D), jnp.float32),
                jax.ShapeDtypeStruct((1, D), jnp.float32),
            ],
        )(h, W1, b1.reshape(1, D), W2, b2.reshape(1, D))
    return pl.pallas_call(
        _mlp_body,
        grid=(GRID,),
        in_specs=in_specs,
        out_specs=_row_spec(),
        out_shape=jax.ShapeDtypeStruct((N, D), jnp.float32),
    )(h, W1, b1.reshape(1, D), W2, b2.reshape(1, D))


def _bn_proj(z2, colsum, colsumsq, gamma, beta, proj_W, proj_b, prelu_a):
    return pl.pallas_call(
        _bn_proj_body,
        grid=(GRID,),
        in_specs=[
            _row_spec(),
            _full_spec((1, D)),
            _full_spec((1, D)),
            _full_spec((1, D)),
            _full_spec((1, D)),
            _full_spec((D, D)),
            _full_spec((1, D)),
            _full_spec((1, 1)),
        ],
        out_specs=[_row_spec(), _row_spec()],
        out_shape=[
            jax.ShapeDtypeStruct((N, D), jnp.float32),
            jax.ShapeDtypeStruct((N, D), jnp.float32),
        ],
    )(z2, colsum, colsumsq, gamma.reshape(1, D), beta.reshape(1, D),
      proj_W, proj_b.reshape(1, D), prelu_a.reshape(1, 1))


def kernel(x, edge_index, W1_0, b1_0, W2_0, b2_0, W1_1, b1_1, W2_1, b2_1,
           bn_gamma, bn_beta, proj_W, proj_b, prelu_a):
    pad = E_PAD - E
    src = jnp.concatenate([edge_index[0], jnp.zeros((pad,), jnp.int32)])
    dst = jnp.concatenate([edge_index[1], jnp.full((pad,), N, jnp.int32)])
    src = src.reshape(IDX_ROWS, CHUNK)
    dst = dst.reshape(IDX_ROWS, CHUNK)
    h1 = _agg(x, src, dst)
    z1 = _mlp(h1, W1_0, b1_0, W2_0, b2_0, with_stats=False)
    h2 = _agg(z1, src, dst)
    z2, colsum, colsumsq = _mlp(h2, W1_1, b1_1, W2_1, b2_1, with_stats=True)
    z, p = _bn_proj(z2, colsum, colsumsq, bn_gamma, bn_beta,
                    proj_W, proj_b, prelu_a)
    return (z, p)


# speedup vs baseline: 1.3263x; 1.3263x over previous
import jax
import jax.numpy as jnp
from jax import lax
from jax.experimental import pallas as pl
from jax.experimental.pallas import tpu as pltpu
from jax.experimental.pallas import tpu_sc as plsc

N = 10000
E = 320000
D = 128

NC = 2
NS = 16
CHUNK = 32
N_PAD = 10240
HALF = N_PAD // 2
E_PAD = 327680
IDX_ROWS = E_PAD // CHUNK
OUTER = IDX_ROWS // NS // 8
ACC_ROWS = HALF + 1
DUMP = HALF


OUTERS = 40
NBLK = HALF // 128


def _agg_body(x_hbm, pk_hbm, out_hbm, xs, acc, cb, stag0, stag1, *rest):
    d1d = rest[:2]
    gs0, gs1 = rest[2:]
    c = lax.axis_index("c")
    s = lax.axis_index("s")
    last = s == NS - 1

    @pl.when(jnp.logical_not(last))
    def _():
        pltpu.sync_copy(x_hbm.at[pl.ds(s * 632, 632)],
                        xs.at[pl.ds(s * 632, 632)])

    @pl.when(last)
    def _():
        pltpu.sync_copy(x_hbm.at[pl.ds(9480, 520)], xs.at[pl.ds(9480, 520)])

    def init_block(k):
        pltpu.sync_copy(x_hbm.at[pl.ds((2 * k + c) * 128, 128)],
                        acc.at[pl.ds(k * 128, 128)])

    @pl.when(s < 8)
    def _():
        for t in range(3):
            init_block(s * 3 + t)

    @pl.when(s >= 8)
    def _():
        init_block(24 + (s - 8) * 2)

    @pl.when((s >= 8) & (s != 15))
    def _():
        init_block(24 + (s - 8) * 2 + 1)

    @pl.when((s == 15) & (c == 0))
    def _():
        pltpu.sync_copy(x_hbm.at[pl.ds(9984, 16)],
                        acc.at[pl.ds(39 * 128, 16)])

    plsc.subcore_barrier()

    bufs = (stag0, stag1)
    gsems = (gs0, gs1)
    dump = 5104 - 24 * c

    def gidx(j):
        return cb.at[j // 4, pl.ds((j % 4) * 32, 32)]

    def outer(i, carry):
        rb = (s * OUTERS + i) * 8
        pltpu.sync_copy(pk_hbm.at[pl.ds(rb, 8)], cb)

        def rewrite(j):
            jr, q = 4 + j // 4, j % 4
            for g in range(2):
                dv = cb[jr, pl.ds(q * 32 + g * 16, 16)]
                own = ((dv >> 7) & 1) == c
                loc = ((dv >> 8) << 7) | (dv & 127)
                d1d[j % 2][pl.ds(g * 16, 16)] = jnp.where(own, loc, dump)

        g = [pltpu.async_copy(xs.at[gidx(0)], stag0, gs0),
             pltpu.async_copy(xs.at[gidx(1)], stag1, gs1)]
        sd = [None] * 16
        for j in range(16):
            b = j % 2
            g[j].wait()
            rewrite(j)
            sd[j] = pltpu.async_copy(bufs[b], acc.at[d1d[b]], gsems[b],
                                     add=True)
            if 1 <= j < 15:
                sd[j - 1].wait()
                g.append(pltpu.async_copy(xs.at[gidx(j + 1)],
                                          bufs[(j + 1) % 2],
                                          gsems[(j + 1) % 2]))
        sd[14].wait()
        sd[15].wait()
        return carry

    lax.fori_loop(0, OUTERS, outer, 0)
    plsc.subcore_barrier()

    def out_block(k):
        pltpu.sync_copy(acc.at[pl.ds(k * 128, 128)],
                        out_hbm.at[pl.ds((2 * k + c) * 128, 128)])

    @pl.when(s < 8)
    def _():
        for t in range(3):
            out_block(s * 3 + t)

    @pl.when(s >= 8)
    def _():
        for t in range(2):
            out_block(24 + (s - 8) * 2 + t)


_agg = pl.kernel(
    _agg_body,
    out_type=jax.ShapeDtypeStruct((N_PAD, D), jnp.float32),
    mesh=plsc.VectorSubcoreMesh(core_axis_name="c", subcore_axis_name="s"),
    scratch_types=[
        pltpu.VMEM_SHARED((N, D), jnp.float32),
        pltpu.VMEM_SHARED((HALF, D), jnp.float32),
        pltpu.VMEM((8, 128), jnp.int32),
        pltpu.VMEM((CHUNK, D), jnp.float32),
        pltpu.VMEM((CHUNK, D), jnp.float32),
    ] + [pltpu.VMEM((CHUNK,), jnp.int32) for _ in range(2)] + [
        pltpu.SemaphoreType.DMA,
        pltpu.SemaphoreType.DMA,
    ],
)


BLK = 1000
GRID = N // BLK


def _mlp_body(h_ref, W1_ref, b1_ref, W2_ref, b2_ref, out_ref):
    h = jnp.dot(h_ref[:], W1_ref[:], preferred_element_type=jnp.float32)
    h = jnp.maximum(h + b1_ref[:], 0.0)
    h = jnp.dot(h, W2_ref[:], preferred_element_type=jnp.float32) + b2_ref[:]
    out_ref[:] = jnp.maximum(h, 0.0)


def _mlp2_body(h_ref, W1_ref, b1_ref, W2_ref, b2_ref,
               out_ref, sum_ref, sumsq_ref):
    h = jnp.dot(h_ref[:], W1_ref[:], preferred_element_type=jnp.float32)
    h = jnp.maximum(h + b1_ref[:], 0.0)
    h = jnp.dot(h, W2_ref[:], preferred_element_type=jnp.float32) + b2_ref[:]
    z = jnp.maximum(h, 0.0)
    out_ref[:] = z
    ps = jnp.sum(z, axis=0, keepdims=True)
    pq = jnp.sum(z * z, axis=0, keepdims=True)

    @pl.when(pl.program_id(0) == 0)
    def _():
        sum_ref[:] = ps
        sumsq_ref[:] = pq

    @pl.when(pl.program_id(0) != 0)
    def _():
        sum_ref[:] = sum_ref[:] + ps
        sumsq_ref[:] = sumsq_ref[:] + pq


def _bn_proj_body(z_ref, sum_ref, sumsq_ref, gamma_ref, beta_ref,
                  pW_ref, pb_ref, a_ref, zo_ref, p_ref):
    mean = sum_ref[:] / N
    var = sumsq_ref[:] / N - mean * mean
    inv = lax.rsqrt(var + 1e-5)
    zn = (z_ref[:] - mean) * (inv * gamma_ref[:]) + beta_ref[:]
    zo_ref[:] = zn
    p = jnp.dot(zn, pW_ref[:], preferred_element_type=jnp.float32) + pb_ref[:]
    p_ref[:] = jnp.where(p >= 0.0, p, a_ref[0, 0] * p)


def _row_spec():
    return pl.BlockSpec((BLK, D), lambda i: (i, 0))


def _full_spec(shape):
    nd = len(shape)
    return pl.BlockSpec(shape, lambda i: (0,) * nd)


def _mlp(h, W1, b1, W2, b2, with_stats):
    in_specs = [
        _row_spec(),
        _full_spec((D, D)),
        _full_spec((1, D)),
        _full_spec((D, D)),
        _full_spec((1, D)),
    ]
    if with_stats:
        return pl.pallas_call(
            _mlp2_body,
            grid=(GRID,),
            in_specs=in_specs,
            out_specs=[_row_spec(), _full_spec((1, D)), _full_spec((1, D))],
            out_shape=[
                jax.ShapeDtypeStruct((N, D), jnp.float32),
                jax.ShapeDtypeStruct((1, D), jnp.float32),
                jax.ShapeDtypeStruct((1, D), jnp.float32),
            ],
        )(h, W1, b1.reshape(1, D), W2, b2.reshape(1, D))
    return pl.pallas_call(
        _mlp_body,
        grid=(GRID,),
        in_specs=in_specs,
        out_specs=_row_spec(),
        out_shape=jax.ShapeDtypeStruct((N, D), jnp.float32),
    )(h, W1, b1.reshape(1, D), W2, b2.reshape(1, D))


def _bn_proj(z2, colsum, colsumsq, gamma, beta, proj_W, proj_b, prelu_a):
    return pl.pallas_call(
        _bn_proj_body,
        grid=(GRID,),
        in_specs=[
            _row_spec(),
            _full_spec((1, D)),
            _full_spec((1, D)),
            _full_spec((1, D)),
            _full_spec((1, D)),
            _full_spec((D, D)),
            _full_spec((1, D)),
            _full_spec((1, 1)),
        ],
        out_specs=[_row_spec(), _row_spec()],
        out_shape=[
            jax.ShapeDtypeStruct((N, D), jnp.float32),
            jax.ShapeDtypeStruct((N, D), jnp.float32),
        ],
    )(z2, colsum, colsumsq, gamma.reshape(1, D), beta.reshape(1, D),
      proj_W, proj_b.reshape(1, D), prelu_a.reshape(1, 1))


def kernel(x, edge_index, W1_0, b1_0, W2_0, b2_0, W1_1, b1_1, W2_1, b2_1,
           bn_gamma, bn_beta, proj_W, proj_b, prelu_a):
    pad = E_PAD - E
    src = jnp.concatenate([edge_index[0], jnp.zeros((pad,), jnp.int32)])
    dst = jnp.concatenate([edge_index[1], jnp.full((pad,), N, jnp.int32)])
    srcr = src.reshape(NS, OUTERS, 4, 128)
    dstr = dst.reshape(NS, OUTERS, 4, 128)
    pk = jnp.concatenate([srcr, dstr], axis=2).reshape(NS * OUTERS * 8, 128)

    h1 = _agg(x, pk)
    z1 = _mlp(h1, W1_0, b1_0, W2_0, b2_0, with_stats=False)
    h2 = _agg(z1, pk)
    z2, colsum, colsumsq = _mlp(h2, W1_1, b1_1, W2_1, b2_1, with_stats=True)
    z, p = _bn_proj(z2, colsum, colsumsq, bn_gamma, bn_beta,
                    proj_W, proj_b, prelu_a)
    return (z, p)


# speedup vs baseline: 1.3611x; 1.0263x over previous
import jax
import jax.numpy as jnp
from jax import lax
from jax.experimental import pallas as pl
from jax.experimental.pallas import tpu as pltpu
from jax.experimental.pallas import tpu_sc as plsc

N = 10000
E = 320000
D = 128

NC = 2
NS = 16
CHUNK = 32
N_PAD = 10240
HALF = N_PAD // 2
E_PAD = 327680
IDX_ROWS = E_PAD // CHUNK
OUTER = IDX_ROWS // NS // 8
ACC_ROWS = HALF + 1
DUMP = HALF


OUTERS = 40
NBLK = HALF // 128


def _agg_body(x_hbm, pk_hbm, out_hbm, xs, acc, cb, stag0, stag1, *rest):
    d1d = rest[:2]
    gs0, gs1, csem = rest[2:]
    c = lax.axis_index("c")
    s = lax.axis_index("s")
    last = s == NS - 1

    @pl.when(jnp.logical_not(last))
    def _():
        pltpu.sync_copy(x_hbm.at[pl.ds(s * 632, 632)],
                        xs.at[pl.ds(s * 632, 632)])

    @pl.when(last)
    def _():
        pltpu.sync_copy(x_hbm.at[pl.ds(9480, 520)], xs.at[pl.ds(9480, 520)])

    def init_block(k):
        pltpu.sync_copy(x_hbm.at[pl.ds((2 * k + c) * 128, 128)],
                        acc.at[pl.ds(k * 128, 128)])

    @pl.when(s < 8)
    def _():
        for t in range(3):
            init_block(s * 3 + t)

    @pl.when(s >= 8)
    def _():
        init_block(24 + (s - 8) * 2)

    @pl.when((s >= 8) & (s != 15))
    def _():
        init_block(24 + (s - 8) * 2 + 1)

    @pl.when((s == 15) & (c == 0))
    def _():
        pltpu.sync_copy(x_hbm.at[pl.ds(9984, 16)],
                        acc.at[pl.ds(39 * 128, 16)])

    plsc.subcore_barrier()

    bufs = (stag0, stag1)
    gsems = (gs0, gs1)
    dump = 5104 - 24 * c

    def gidx(j):
        return cb.at[j // 4, pl.ds((j % 4) * 32, 32)]

    pltpu.async_copy(pk_hbm.at[pl.ds(s * OUTERS * 8, 8)], cb, csem)

    def outer(i, carry):
        pltpu.make_async_copy(pk_hbm.at[pl.ds(0, 8)], cb, csem).wait()

        def rewrite(j):
            jr, q = 4 + j // 4, j % 4
            for g in range(2):
                dv = cb[jr, pl.ds(q * 32 + g * 16, 16)]
                own = ((dv >> 7) & 1) == c
                loc = ((dv >> 8) << 7) | (dv & 127)
                d1d[j % 2][pl.ds(g * 16, 16)] = jnp.where(own, loc, dump)

        g = [pltpu.async_copy(xs.at[gidx(0)], stag0, gs0),
             pltpu.async_copy(xs.at[gidx(1)], stag1, gs1)]
        sd = [None] * 16
        for j in range(16):
            b = j % 2
            g[j].wait()
            rewrite(j)
            sd[j] = pltpu.async_copy(bufs[b], acc.at[d1d[b]], gsems[b],
                                     add=True)
            if 1 <= j < 15:
                sd[j - 1].wait()
                g.append(pltpu.async_copy(xs.at[gidx(j + 1)],
                                          bufs[(j + 1) % 2],
                                          gsems[(j + 1) % 2]))
        @pl.when(i < OUTERS - 1)
        def _():
            rb = (s * OUTERS + i + 1) * 8
            pltpu.async_copy(pk_hbm.at[pl.ds(rb, 8)], cb, csem)
        sd[14].wait()
        sd[15].wait()
        return carry

    lax.fori_loop(0, OUTERS, outer, 0)
    plsc.subcore_barrier()

    def out_block(k):
        pltpu.sync_copy(acc.at[pl.ds(k * 128, 128)],
                        out_hbm.at[pl.ds((2 * k + c) * 128, 128)])

    @pl.when(s < 8)
    def _():
        for t in range(3):
            out_block(s * 3 + t)

    @pl.when(s >= 8)
    def _():
        for t in range(2):
            out_block(24 + (s - 8) * 2 + t)


_agg = pl.kernel(
    _agg_body,
    out_type=jax.ShapeDtypeStruct((N_PAD, D), jnp.float32),
    mesh=plsc.VectorSubcoreMesh(core_axis_name="c", subcore_axis_name="s"),
    scratch_types=[
        pltpu.VMEM_SHARED((N, D), jnp.float32),
        pltpu.VMEM_SHARED((HALF, D), jnp.float32),
        pltpu.VMEM((8, 128), jnp.int32),
        pltpu.VMEM((CHUNK, D), jnp.float32),
        pltpu.VMEM((CHUNK, D), jnp.float32),
    ] + [pltpu.VMEM((CHUNK,), jnp.int32) for _ in range(2)] + [
        pltpu.SemaphoreType.DMA,
        pltpu.SemaphoreType.DMA,
        pltpu.SemaphoreType.DMA,
    ],
)


BLK = 1000
GRID = N // BLK


def _mlp_body(h_ref, W1_ref, b1_ref, W2_ref, b2_ref, out_ref):
    h = jnp.dot(h_ref[:], W1_ref[:], preferred_element_type=jnp.float32)
    h = jnp.maximum(h + b1_ref[:], 0.0)
    h = jnp.dot(h, W2_ref[:], preferred_element_type=jnp.float32) + b2_ref[:]
    out_ref[:] = jnp.maximum(h, 0.0)


def _mlp2_body(h_ref, W1_ref, b1_ref, W2_ref, b2_ref,
               out_ref, sum_ref, sumsq_ref):
    h = jnp.dot(h_ref[:], W1_ref[:], preferred_element_type=jnp.float32)
    h = jnp.maximum(h + b1_ref[:], 0.0)
    h = jnp.dot(h, W2_ref[:], preferred_element_type=jnp.float32) + b2_ref[:]
    z = jnp.maximum(h, 0.0)
    out_ref[:] = z
    ps = jnp.sum(z, axis=0, keepdims=True)
    pq = jnp.sum(z * z, axis=0, keepdims=True)

    @pl.when(pl.program_id(0) == 0)
    def _():
        sum_ref[:] = ps
        sumsq_ref[:] = pq

    @pl.when(pl.program_id(0) != 0)
    def _():
        sum_ref[:] = sum_ref[:] + ps
        sumsq_ref[:] = sumsq_ref[:] + pq


def _bn_proj_body(z_ref, sum_ref, sumsq_ref, gamma_ref, beta_ref,
                  pW_ref, pb_ref, a_ref, zo_ref, p_ref):
    mean = sum_ref[:] / N
    var = sumsq_ref[:] / N - mean * mean
    inv = lax.rsqrt(var + 1e-5)
    zn = (z_ref[:] - mean) * (inv * gamma_ref[:]) + beta_ref[:]
    zo_ref[:] = zn
    p = jnp.dot(zn, pW_ref[:], preferred_element_type=jnp.float32) + pb_ref[:]
    p_ref[:] = jnp.where(p >= 0.0, p, a_ref[0, 0] * p)


def _row_spec():
    return pl.BlockSpec((BLK, D), lambda i: (i, 0))


def _full_spec(shape):
    nd = len(shape)
    return pl.BlockSpec(shape, lambda i: (0,) * nd)


def _mlp(h, W1, b1, W2, b2, with_stats):
    in_specs = [
        _row_spec(),
        _full_spec((D, D)),
        _full_spec((1, D)),
        _full_spec((D, D)),
        _full_spec((1, D)),
    ]
    if with_stats:
        return pl.pallas_call(
            _mlp2_body,
            grid=(GRID,),
            in_specs=in_specs,
            out_specs=[_row_spec(), _full_spec((1, D)), _full_spec((1, D))],
            out_shape=[
                jax.ShapeDtypeStruct((N, D), jnp.float32),
                jax.ShapeDtypeStruct((1, D), jnp.float32),
                jax.ShapeDtypeStruct((1, D), jnp.float32),
            ],
        )(h, W1, b1.reshape(1, D), W2, b2.reshape(1, D))
    return pl.pallas_call(
        _mlp_body,
        grid=(GRID,),
        in_specs=in_specs,
        out_specs=_row_spec(),
        out_shape=jax.ShapeDtypeStruct((N, D), jnp.float32),
    )(h, W1, b1.reshape(1, D), W2, b2.reshape(1, D))


def _bn_proj(z2, colsum, colsumsq, gamma, beta, proj_W, proj_b, prelu_a):
    return pl.pallas_call(
        _bn_proj_body,
        grid=(GRID,),
        in_specs=[
            _row_spec(),
            _full_spec((1, D)),
            _full_spec((1, D)),
            _full_spec((1, D)),
            _full_spec((1, D)),
            _full_spec((D, D)),
            _full_spec((1, D)),
            _full_spec((1, 1)),
        ],
        out_specs=[_row_spec(), _row_spec()],
        out_shape=[
            jax.ShapeDtypeStruct((N, D), jnp.float32),
            jax.ShapeDtypeStruct((N, D), jnp.float32),
        ],
    )(z2, colsum, colsumsq, gamma.reshape(1, D), beta.reshape(1, D),
      proj_W, proj_b.reshape(1, D), prelu_a.reshape(1, 1))


def kernel(x, edge_index, W1_0, b1_0, W2_0, b2_0, W1_1, b1_1, W2_1, b2_1,
           bn_gamma, bn_beta, proj_W, proj_b, prelu_a):
    pad = E_PAD - E
    src = jnp.concatenate([edge_index[0], jnp.zeros((pad,), jnp.int32)])
    dst = jnp.concatenate([edge_index[1], jnp.full((pad,), N, jnp.int32)])
    srcr = src.reshape(NS, OUTERS, 4, 128)
    dstr = dst.reshape(NS, OUTERS, 4, 128)
    pk = jnp.concatenate([srcr, dstr], axis=2).reshape(NS * OUTERS * 8, 128)

    h1 = _agg(x, pk)
    z1 = _mlp(h1, W1_0, b1_0, W2_0, b2_0, with_stats=False)
    h2 = _agg(z1, pk)
    z2, colsum, colsumsq = _mlp(h2, W1_1, b1_1, W2_1, b2_1, with_stats=True)
    z, p = _bn_proj(z2, colsum, colsumsq, bn_gamma, bn_beta,
                    proj_W, proj_b, prelu_a)
    return (z, p)


# speedup vs baseline: 1.4492x; 1.0647x over previous
import jax
import jax.numpy as jnp
from jax import lax
from jax.experimental import pallas as pl
from jax.experimental.pallas import tpu as pltpu
from jax.experimental.pallas import tpu_sc as plsc

N = 10000
E = 320000
D = 128

NC = 2
NS = 16
CHUNK = 32
N_PAD = 10240
HALF = N_PAD // 2
E_PAD = 327680
IDX_ROWS = E_PAD // CHUNK
OUTER = IDX_ROWS // NS // 8
ACC_ROWS = HALF + 1
DUMP = HALF


OUTERS = 40
NBLK = HALF // 128


def _agg_body(x_hbm, pk_hbm, out_hbm, xs, acc, cb, stag0, stag1, *rest):
    d1d = rest[:2]
    gs0, gs1, csem = rest[2:]
    c = lax.axis_index("c")
    s = lax.axis_index("s")
    last = s == NS - 1

    @pl.when(jnp.logical_not(last))
    def _():
        pltpu.sync_copy(x_hbm.at[pl.ds(s * 632, 632)],
                        xs.at[pl.ds(s * 632, 632)])

    @pl.when(last)
    def _():
        pltpu.sync_copy(x_hbm.at[pl.ds(9480, 520)], xs.at[pl.ds(9480, 520)])

    def init_block(k):
        pltpu.sync_copy(x_hbm.at[pl.ds((2 * k + c) * 128, 128)],
                        acc.at[pl.ds(k * 128, 128)])

    @pl.when(s < 8)
    def _():
        for t in range(3):
            init_block(s * 3 + t)

    @pl.when(s >= 8)
    def _():
        init_block(24 + (s - 8) * 2)

    @pl.when((s >= 8) & (s != 15))
    def _():
        init_block(24 + (s - 8) * 2 + 1)

    @pl.when((s == 15) & (c == 0))
    def _():
        pltpu.sync_copy(x_hbm.at[pl.ds(9984, 16)],
                        acc.at[pl.ds(39 * 128, 16)])

    plsc.subcore_barrier()

    bufs = (stag0, stag1)
    gsems = (gs0, gs1)
    iota16 = lax.iota(jnp.int32, 16)
    dump = (5008 - 16 * c) + (s << 2) + (iota16 & 3)

    def gidx(j):
        return cb.at[j // 4, pl.ds((j % 4) * 32, 32)]

    pltpu.async_copy(pk_hbm.at[pl.ds(s * OUTERS * 8, 8)], cb, csem)

    def outer(i, carry):
        pltpu.make_async_copy(pk_hbm.at[pl.ds(0, 8)], cb, csem).wait()

        def rewrite(j):
            jr, q = 4 + j // 4, j % 4
            for g in range(2):
                dv = cb[jr, pl.ds(q * 32 + g * 16, 16)]
                own = ((dv >> 7) & 1) == c
                loc = ((dv >> 8) << 7) | (dv & 127)
                d1d[j % 2][pl.ds(g * 16, 16)] = jnp.where(own, loc, dump)

        g = [pltpu.async_copy(xs.at[gidx(0)], stag0, gs0),
             pltpu.async_copy(xs.at[gidx(1)], stag1, gs1)]
        sd = [None] * 16
        for j in range(16):
            b = j % 2
            g[j].wait()
            rewrite(j)
            sd[j] = pltpu.async_copy(bufs[b], acc.at[d1d[b]], gsems[b],
                                     add=True)
            if 1 <= j < 15:
                sd[j - 1].wait()
                g.append(pltpu.async_copy(xs.at[gidx(j + 1)],
                                          bufs[(j + 1) % 2],
                                          gsems[(j + 1) % 2]))
        @pl.when(i < OUTERS - 1)
        def _():
            rb = (s * OUTERS + i + 1) * 8
            pltpu.async_copy(pk_hbm.at[pl.ds(rb, 8)], cb, csem)
        sd[14].wait()
        sd[15].wait()
        return carry

    lax.fori_loop(0, OUTERS, outer, 0)
    plsc.subcore_barrier()

    def out_block(k):
        pltpu.sync_copy(acc.at[pl.ds(k * 128, 128)],
                        out_hbm.at[pl.ds((2 * k + c) * 128, 128)])

    @pl.when(s < 8)
    def _():
        for t in range(3):
            out_block(s * 3 + t)

    @pl.when(s >= 8)
    def _():
        for t in range(2):
            out_block(24 + (s - 8) * 2 + t)


_agg = pl.kernel(
    _agg_body,
    out_type=jax.ShapeDtypeStruct((N_PAD, D), jnp.float32),
    mesh=plsc.VectorSubcoreMesh(core_axis_name="c", subcore_axis_name="s"),
    scratch_types=[
        pltpu.VMEM_SHARED((N, D), jnp.float32),
        pltpu.VMEM_SHARED((HALF, D), jnp.float32),
        pltpu.VMEM((8, 128), jnp.int32),
        pltpu.VMEM((CHUNK, D), jnp.float32),
        pltpu.VMEM((CHUNK, D), jnp.float32),
    ] + [pltpu.VMEM((CHUNK,), jnp.int32) for _ in range(2)] + [
        pltpu.SemaphoreType.DMA,
        pltpu.SemaphoreType.DMA,
        pltpu.SemaphoreType.DMA,
    ],
)


BLK = 1000
GRID = N // BLK


def _mlp_body(h_ref, W1_ref, b1_ref, W2_ref, b2_ref, out_ref):
    h = jnp.dot(h_ref[:], W1_ref[:], preferred_element_type=jnp.float32)
    h = jnp.maximum(h + b1_ref[:], 0.0)
    h = jnp.dot(h, W2_ref[:], preferred_element_type=jnp.float32) + b2_ref[:]
    out_ref[:] = jnp.maximum(h, 0.0)


def _mlp2_body(h_ref, W1_ref, b1_ref, W2_ref, b2_ref,
               out_ref, sum_ref, sumsq_ref):
    h = jnp.dot(h_ref[:], W1_ref[:], preferred_element_type=jnp.float32)
    h = jnp.maximum(h + b1_ref[:], 0.0)
    h = jnp.dot(h, W2_ref[:], preferred_element_type=jnp.float32) + b2_ref[:]
    z = jnp.maximum(h, 0.0)
    out_ref[:] = z
    ps = jnp.sum(z, axis=0, keepdims=True)
    pq = jnp.sum(z * z, axis=0, keepdims=True)

    @pl.when(pl.program_id(0) == 0)
    def _():
        sum_ref[:] = ps
        sumsq_ref[:] = pq

    @pl.when(pl.program_id(0) != 0)
    def _():
        sum_ref[:] = sum_ref[:] + ps
        sumsq_ref[:] = sumsq_ref[:] + pq


def _bn_proj_body(z_ref, sum_ref, sumsq_ref, gamma_ref, beta_ref,
                  pW_ref, pb_ref, a_ref, zo_ref, p_ref):
    mean = sum_ref[:] / N
    var = sumsq_ref[:] / N - mean * mean
    inv = lax.rsqrt(var + 1e-5)
    zn = (z_ref[:] - mean) * (inv * gamma_ref[:]) + beta_ref[:]
    zo_ref[:] = zn
    p = jnp.dot(zn, pW_ref[:], preferred_element_type=jnp.float32) + pb_ref[:]
    p_ref[:] = jnp.where(p >= 0.0, p, a_ref[0, 0] * p)


def _row_spec():
    return pl.BlockSpec((BLK, D), lambda i: (i, 0))


def _full_spec(shape):
    nd = len(shape)
    return pl.BlockSpec(shape, lambda i: (0,) * nd)


def _mlp(h, W1, b1, W2, b2, with_stats):
    in_specs = [
        _row_spec(),
        _full_spec((D, D)),
        _full_spec((1, D)),
        _full_spec((D, D)),
        _full_spec((1, D)),
    ]
    if with_stats:
        return pl.pallas_call(
            _mlp2_body,
            grid=(GRID,),
            in_specs=in_specs,
            out_specs=[_row_spec(), _full_spec((1, D)), _full_spec((1, D))],
            out_shape=[
                jax.ShapeDtypeStruct((N, D), jnp.float32),
                jax.ShapeDtypeStruct((1, D), jnp.float32),
                jax.ShapeDtypeStruct((1, D), jnp.float32),
            ],
        )(h, W1, b1.reshape(1, D), W2, b2.reshape(1, D))
    return pl.pallas_call(
        _mlp_body,
        grid=(GRID,),
        in_specs=in_specs,
        out_specs=_row_spec(),
        out_shape=jax.ShapeDtypeStruct((N, D), jnp.float32),
    )(h, W1, b1.reshape(1, D), W2, b2.reshape(1, D))


def _bn_proj(z2, colsum, colsumsq, gamma, beta, proj_W, proj_b, prelu_a):
    return pl.pallas_call(
        _bn_proj_body,
        grid=(GRID,),
        in_specs=[
            _row_spec(),
            _full_spec((1, D)),
            _full_spec((1, D)),
            _full_spec((1, D)),
            _full_spec((1, D)),
            _full_spec((D, D)),
            _full_spec((1, D)),
            _full_spec((1, 1)),
        ],
        out_specs=[_row_spec(), _row_spec()],
        out_shape=[
            jax.ShapeDtypeStruct((N, D), jnp.float32),
            jax.ShapeDtypeStruct((N, D), jnp.float32),
        ],
    )(z2, colsum, colsumsq, gamma.reshape(1, D), beta.reshape(1, D),
      proj_W, proj_b.reshape(1, D), prelu_a.reshape(1, 1))


def kernel(x, edge_index, W1_0, b1_0, W2_0, b2_0, W1_1, b1_1, W2_1, b2_1,
           bn_gamma, bn_beta, proj_W, proj_b, prelu_a):
    pad = E_PAD - E
    src = jnp.concatenate([edge_index[0], jnp.zeros((pad,), jnp.int32)])
    dst = jnp.concatenate([edge_index[1], jnp.full((pad,), N, jnp.int32)])
    srcr = src.reshape(NS, OUTERS, 4, 128)
    dstr = dst.reshape(NS, OUTERS, 4, 128)
    pk = jnp.concatenate([srcr, dstr], axis=2).reshape(NS * OUTERS * 8, 128)

    h1 = _agg(x, pk)
    z1 = _mlp(h1, W1_0, b1_0, W2_0, b2_0, with_stats=False)
    h2 = _agg(z1, pk)
    z2, colsum, colsumsq = _mlp(h2, W1_1, b1_1, W2_1, b2_1, with_stats=True)
    z, p = _bn_proj(z2, colsum, colsumsq, bn_gamma, bn_beta,
                    proj_W, proj_b, prelu_a)
    return (z, p)


# speedup vs baseline: 1.4502x; 1.0007x over previous
import jax
import jax.numpy as jnp
from jax import lax
from jax.experimental import pallas as pl
from jax.experimental.pallas import tpu as pltpu
from jax.experimental.pallas import tpu_sc as plsc

N = 10000
E = 320000
D = 128

NC = 2
NS = 16
CHUNK = 32
N_PAD = 10240
HALF = N_PAD // 2
E_PAD = 327680
IDX_ROWS = E_PAD // CHUNK
OUTER = IDX_ROWS // NS // 8
ACC_ROWS = HALF + 1
DUMP = HALF


OUTERS = 40
NBLK = HALF // 128


def _agg_body(x_hbm, pk_hbm, out_hbm, xs, acc, cb, stag0, stag1, *rest):
    d1d = rest[:2]
    gs0, gs1, csem = rest[2:]
    c = lax.axis_index("c")
    s = lax.axis_index("s")
    last = s == NS - 1

    @pl.when(jnp.logical_not(last))
    def _():
        pltpu.sync_copy(x_hbm.at[pl.ds(s * 632, 632)],
                        xs.at[pl.ds(s * 632, 632)])

    @pl.when(last)
    def _():
        pltpu.sync_copy(x_hbm.at[pl.ds(9480, 520)], xs.at[pl.ds(9480, 520)])

    def init_block(k):
        pltpu.sync_copy(x_hbm.at[pl.ds((2 * k + c) * 128, 128)],
                        acc.at[pl.ds(k * 128, 128)])

    @pl.when(s < 8)
    def _():
        for t in range(3):
            init_block(s * 3 + t)

    @pl.when(s >= 8)
    def _():
        init_block(24 + (s - 8) * 2)

    @pl.when((s >= 8) & (s != 15))
    def _():
        init_block(24 + (s - 8) * 2 + 1)

    @pl.when((s == 15) & (c == 0))
    def _():
        pltpu.sync_copy(x_hbm.at[pl.ds(9984, 16)],
                        acc.at[pl.ds(39 * 128, 16)])

    plsc.subcore_barrier()

    bufs = (stag0, stag1)
    gsems = (gs0, gs1)
    iota16 = lax.iota(jnp.int32, 16)
    dump = (5008 - 16 * c) + (s << 2) + (iota16 & 3)

    def gidx(j):
        return cb.at[j // 4, pl.ds((j % 4) * 32, 32)]

    pltpu.async_copy(pk_hbm.at[pl.ds(s * OUTERS * 8, 8)], cb, csem)

    def outer(i, carry):
        pltpu.make_async_copy(pk_hbm.at[pl.ds(0, 8)], cb, csem).wait()

        def rewrite(j):
            jr, q = 4 + j // 4, j % 4
            for g in range(2):
                dv = cb[jr, pl.ds(q * 32 + g * 16, 16)]
                own = ((dv >> 7) & 1) == c
                loc = ((dv >> 8) << 7) | (dv & 127)
                d1d[j % 2][pl.ds(g * 16, 16)] = jnp.where(own, loc, dump)

        g = [pltpu.async_copy(xs.at[gidx(0)], stag0, gs0),
             pltpu.async_copy(xs.at[gidx(1)], stag1, gs1)]
        sd = [None] * 16
        for j in range(16):
            b = j % 2
            g[j].wait()
            rewrite(j)
            sd[j] = pltpu.async_copy(bufs[b], acc.at[d1d[b]], gsems[b],
                                     add=True)
            if 1 <= j < 15:
                sd[j - 1].wait()
                g.append(pltpu.async_copy(xs.at[gidx(j + 1)],
                                          bufs[(j + 1) % 2],
                                          gsems[(j + 1) % 2]))
        @pl.when(i < OUTERS - 1)
        def _():
            rb = (s * OUTERS + i + 1) * 8
            pltpu.async_copy(pk_hbm.at[pl.ds(rb, 8)], cb, csem)
        sd[14].wait()
        sd[15].wait()
        return carry

    lax.fori_loop(0, OUTERS, outer, 0)
    plsc.subcore_barrier()

    def out_block(k):
        pltpu.sync_copy(acc.at[pl.ds(k * 128, 128)],
                        out_hbm.at[pl.ds((2 * k + c) * 128, 128)])

    @pl.when(s < 8)
    def _():
        for t in range(3):
            out_block(s * 3 + t)

    @pl.when(s >= 8)
    def _():
        for t in range(2):
            out_block(24 + (s - 8) * 2 + t)


_agg = pl.kernel(
    _agg_body,
    out_type=jax.ShapeDtypeStruct((N_PAD, D), jnp.float32),
    mesh=plsc.VectorSubcoreMesh(core_axis_name="c", subcore_axis_name="s"),
    scratch_types=[
        pltpu.VMEM_SHARED((N, D), jnp.float32),
        pltpu.VMEM_SHARED((HALF, D), jnp.float32),
        pltpu.VMEM((8, 128), jnp.int32),
        pltpu.VMEM((CHUNK, D), jnp.float32),
        pltpu.VMEM((CHUNK, D), jnp.float32),
    ] + [pltpu.VMEM((CHUNK,), jnp.int32) for _ in range(2)] + [
        pltpu.SemaphoreType.DMA,
        pltpu.SemaphoreType.DMA,
        pltpu.SemaphoreType.DMA,
    ],
)


BLK = 1000
GRID = N // BLK


def _mlp_body(h_ref, W1_ref, b1_ref, W2_ref, b2_ref, out_ref):
    h = jnp.dot(h_ref[:], W1_ref[:], preferred_element_type=jnp.float32)
    h = jnp.maximum(h + b1_ref[:], 0.0)
    h = jnp.dot(h, W2_ref[:], preferred_element_type=jnp.float32) + b2_ref[:]
    out_ref[:] = jnp.maximum(h, 0.0)


def _tail_body(h_ref, W1_ref, b1_ref, W2_ref, b2_ref, gamma_ref, beta_ref,
               pW_ref, pb_ref, a_ref, zo_ref, p_ref,
               z2_scr, sum_scr, sumsq_scr):
    i = pl.program_id(0)

    @pl.when(i < GRID)
    def _():
        h = jnp.dot(h_ref[:], W1_ref[:], preferred_element_type=jnp.float32)
        h = jnp.maximum(h + b1_ref[:], 0.0)
        h = (jnp.dot(h, W2_ref[:], preferred_element_type=jnp.float32)
             + b2_ref[:])
        z = jnp.maximum(h, 0.0)
        z2_scr[pl.ds(i * BLK, BLK), :] = z
        ps = jnp.sum(z, axis=0, keepdims=True)
        pq = jnp.sum(z * z, axis=0, keepdims=True)

        @pl.when(i == 0)
        def _():
            sum_scr[:] = ps
            sumsq_scr[:] = pq

        @pl.when(i != 0)
        def _():
            sum_scr[:] = sum_scr[:] + ps
            sumsq_scr[:] = sumsq_scr[:] + pq

    @pl.when(i >= GRID)
    def _():
        j = i - GRID
        mean = sum_scr[:] / N
        var = sumsq_scr[:] / N - mean * mean
        inv = lax.rsqrt(var + 1e-5)
        z2 = z2_scr[pl.ds(j * BLK, BLK), :]
        zn = (z2 - mean) * (inv * gamma_ref[:]) + beta_ref[:]
        zo_ref[:] = zn
        p = (jnp.dot(zn, pW_ref[:], preferred_element_type=jnp.float32)
             + pb_ref[:])
        p_ref[:] = jnp.where(p >= 0.0, p, a_ref[0, 0] * p)


def _row_spec():
    return pl.BlockSpec((BLK, D), lambda i: (i, 0))


def _full_spec(shape):
    nd = len(shape)
    return pl.BlockSpec(shape, lambda i: (0,) * nd)


def _mlp(h, W1, b1, W2, b2):
    return pl.pallas_call(
        _mlp_body,
        grid=(GRID,),
        in_specs=[
            _row_spec(),
            _full_spec((D, D)),
            _full_spec((1, D)),
            _full_spec((D, D)),
            _full_spec((1, D)),
        ],
        out_specs=_row_spec(),
        out_shape=jax.ShapeDtypeStruct((N, D), jnp.float32),
    )(h, W1, b1.reshape(1, D), W2, b2.reshape(1, D))


def _tail(h, W1, b1, W2, b2, gamma, beta, proj_W, proj_b, prelu_a):
    blk = pl.BlockSpec((BLK, D), lambda i: (i % GRID, 0))
    return pl.pallas_call(
        _tail_body,
        grid=(2 * GRID,),
        in_specs=[
            blk,
            _full_spec((D, D)),
            _full_spec((1, D)),
            _full_spec((D, D)),
            _full_spec((1, D)),
            _full_spec((1, D)),
            _full_spec((1, D)),
            _full_spec((D, D)),
            _full_spec((1, D)),
            _full_spec((1, 1)),
        ],
        out_specs=[blk, blk],
        out_shape=[
            jax.ShapeDtypeStruct((N, D), jnp.float32),
            jax.ShapeDtypeStruct((N, D), jnp.float32),
        ],
        scratch_shapes=[
            pltpu.VMEM((N, D), jnp.float32),
            pltpu.VMEM((1, D), jnp.float32),
            pltpu.VMEM((1, D), jnp.float32),
        ],
    )(h, W1, b1.reshape(1, D), W2, b2.reshape(1, D),
      gamma.reshape(1, D), beta.reshape(1, D),
      proj_W, proj_b.reshape(1, D), prelu_a.reshape(1, 1))


def kernel(x, edge_index, W1_0, b1_0, W2_0, b2_0, W1_1, b1_1, W2_1, b2_1,
           bn_gamma, bn_beta, proj_W, proj_b, prelu_a):
    pad = E_PAD - E
    src = jnp.concatenate([edge_index[0], jnp.zeros((pad,), jnp.int32)])
    dst = jnp.concatenate([edge_index[1], jnp.full((pad,), N, jnp.int32)])
    srcr = src.reshape(NS, OUTERS, 4, 128)
    dstr = dst.reshape(NS, OUTERS, 4, 128)
    pk = jnp.concatenate([srcr, dstr], axis=2).reshape(NS * OUTERS * 8, 128)

    h1 = _agg(x, pk)
    z1 = _mlp(h1, W1_0, b1_0, W2_0, b2_0)
    h2 = _agg(z1, pk)
    z, p = _tail(h2, W1_1, b1_1, W2_1, b2_1, bn_gamma, bn_beta,
                 proj_W, proj_b, prelu_a)
    return (z, p)
